# speedup vs baseline: 2.6429x; 2.6429x over previous
import jax
import jax.numpy as jnp
from jax.experimental import pallas as pl
from jax.experimental.pallas import tpu as pltpu

T = 2048
D = 1024
E = 8
K = 2
DFF = 512
TB = 256
NTB = T // TB


def _moe_body(x_ref, gate_ref, w1g_ref, w1u_ref, w2_ref, swg_ref, swu_ref,
              swd_ref, out_ref, xbf_ref, wfull_ref, wg_ref, wu_ref, wd_ref):
    e = pl.program_id(0)

    @pl.when(e == 0)
    def _router():
        x = x_ref[...]
        xbf_ref[...] = x.astype(jnp.bfloat16)
        logits = jax.lax.dot_general(
            x, gate_ref[...], (((1,), (1,)), ((), ())),
            preferred_element_type=jnp.float32)
        m = jnp.max(logits, axis=-1, keepdims=True)
        ex = jnp.exp(logits - m)
        probs = ex / jnp.sum(ex, axis=-1, keepdims=True)
        lane = jax.lax.broadcasted_iota(jnp.int32, (T, E), 1)
        v1 = jnp.max(probs, axis=-1, keepdims=True)
        i1 = jnp.min(jnp.where(probs == v1, lane, E), axis=-1, keepdims=True)
        m1 = lane == i1
        probs2 = jnp.where(m1, -1.0, probs)
        v2 = jnp.max(probs2, axis=-1, keepdims=True)
        i2 = jnp.min(jnp.where(probs2 == v2, lane, E), axis=-1, keepdims=True)
        m2 = lane == i2
        denom = v1 + v2
        wfull_ref[...] = (jnp.where(m1, v1, 0.0) + jnp.where(m2, v2, 0.0)) / denom
        out_ref[...] = jnp.zeros((T, D), jnp.float32)

    @pl.when(e < E)
    def _stage_routed():
        wg_ref[...] = w1g_ref[0].astype(jnp.bfloat16)
        wu_ref[...] = w1u_ref[0].astype(jnp.bfloat16)
        wd_ref[...] = w2_ref[0].astype(jnp.bfloat16)

    @pl.when(e == E)
    def _stage_shared():
        wg_ref[...] = swg_ref[...].astype(jnp.bfloat16)
        wu_ref[...] = swu_ref[...].astype(jnp.bfloat16)
        wd_ref[...] = swd_ref[...].astype(jnp.bfloat16)

    lane = jax.lax.broadcasted_iota(jnp.int32, (TB, E), 1)
    for tb in range(NTB):
        rows = pl.ds(tb * TB, TB)
        xb = xbf_ref[rows, :]
        g = jax.lax.dot_general(xb, wg_ref[...], (((1,), (1,)), ((), ())),
                                preferred_element_type=jnp.float32)
        u = jax.lax.dot_general(xb, wu_ref[...], (((1,), (1,)), ((), ())),
                                preferred_element_type=jnp.float32)
        h = (g * (1.0 / (1.0 + jnp.exp(-g)))) * u
        o = jax.lax.dot_general(h.astype(jnp.bfloat16), wd_ref[...],
                                (((1,), (1,)), ((), ())),
                                preferred_element_type=jnp.float32)
        sel = jnp.sum(jnp.where(lane == e, wfull_ref[rows, :], 0.0),
                      axis=-1, keepdims=True)
        w = jnp.where(e == E, 1.0, sel)
        out_ref[rows, :] += w * o


@jax.jit
def kernel(hidden_states, gate_w, w1_gate, w1_up, w2, sw_gate, sw_up, sw_down):
    grid = (E + 1,)
    out = pl.pallas_call(
        _moe_body,
        grid=grid,
        in_specs=[
            pl.BlockSpec((T, D), lambda e: (0, 0)),
            pl.BlockSpec((E, D), lambda e: (0, 0)),
            pl.BlockSpec((1, DFF, D), lambda e: (jnp.minimum(e, E - 1), 0, 0)),
            pl.BlockSpec((1, DFF, D), lambda e: (jnp.minimum(e, E - 1), 0, 0)),
            pl.BlockSpec((1, D, DFF), lambda e: (jnp.minimum(e, E - 1), 0, 0)),
            pl.BlockSpec((DFF, D), lambda e: (0, 0)),
            pl.BlockSpec((DFF, D), lambda e: (0, 0)),
            pl.BlockSpec((D, DFF), lambda e: (0, 0)),
        ],
        out_specs=pl.BlockSpec((T, D), lambda e: (0, 0)),
        out_shape=jax.ShapeDtypeStruct((T, D), jnp.float32),
        scratch_shapes=[
            pltpu.VMEM((T, D), jnp.bfloat16),
            pltpu.VMEM((T, E), jnp.float32),
            pltpu.VMEM((DFF, D), jnp.bfloat16),
            pltpu.VMEM((DFF, D), jnp.bfloat16),
            pltpu.VMEM((D, DFF), jnp.bfloat16),
        ],
        compiler_params=pltpu.CompilerParams(
            dimension_semantics=("arbitrary",)),
    )(hidden_states, gate_w, w1_gate, w1_up, w2, sw_gate, sw_up, sw_down)
    return out


# speedup vs baseline: 2.7585x; 1.0437x over previous
import jax
import jax.numpy as jnp
from jax.experimental import pallas as pl
from jax.experimental.pallas import tpu as pltpu

T = 2048
D = 1024
E = 8
K = 2
DFF = 512
TB = 512
NTB = T // TB


def _moe_body(x_ref, gate_ref, w1g_ref, w1u_ref, w2_ref, swg_ref, swu_ref,
              swd_ref, out_ref, xbf_ref, wfull_ref, wgu_ref, wd_ref):
    e = pl.program_id(0)

    @pl.when(e == 0)
    def _router():
        x = x_ref[...]
        xbf_ref[...] = x.astype(jnp.bfloat16)
        logits = jax.lax.dot_general(
            x, gate_ref[...], (((1,), (1,)), ((), ())),
            preferred_element_type=jnp.float32)
        m = jnp.max(logits, axis=-1, keepdims=True)
        ex = jnp.exp(logits - m)
        probs = ex / jnp.sum(ex, axis=-1, keepdims=True)
        lane = jax.lax.broadcasted_iota(jnp.int32, (T, E), 1)
        v1 = jnp.max(probs, axis=-1, keepdims=True)
        i1 = jnp.min(jnp.where(probs == v1, lane, E), axis=-1, keepdims=True)
        m1 = lane == i1
        probs2 = jnp.where(m1, -1.0, probs)
        v2 = jnp.max(probs2, axis=-1, keepdims=True)
        i2 = jnp.min(jnp.where(probs2 == v2, lane, E), axis=-1, keepdims=True)
        m2 = lane == i2
        denom = v1 + v2
        wfull_ref[...] = (jnp.where(m1, v1, 0.0) + jnp.where(m2, v2, 0.0)) / denom
        out_ref[...] = jnp.zeros((T, D), jnp.float32)

    @pl.when(e < E)
    def _stage_routed():
        wgu_ref[0:DFF, :] = w1g_ref[0].astype(jnp.bfloat16)
        wgu_ref[DFF:2 * DFF, :] = w1u_ref[0].astype(jnp.bfloat16)
        wd_ref[...] = w2_ref[0].astype(jnp.bfloat16)

    @pl.when(e == E)
    def _stage_shared():
        wgu_ref[0:DFF, :] = swg_ref[...].astype(jnp.bfloat16)
        wgu_ref[DFF:2 * DFF, :] = swu_ref[...].astype(jnp.bfloat16)
        wd_ref[...] = swd_ref[...].astype(jnp.bfloat16)

    lane = jax.lax.broadcasted_iota(jnp.int32, (TB, E), 1)
    for tb in range(NTB):
        rows = pl.ds(tb * TB, TB)
        xb = xbf_ref[rows, :]
        gu = jax.lax.dot_general(xb, wgu_ref[...], (((1,), (1,)), ((), ())),
                                 preferred_element_type=jnp.float32)
        g = gu[:, 0:DFF]
        u = gu[:, DFF:2 * DFF]
        h = (g * (1.0 / (1.0 + jnp.exp(-g)))) * u
        o = jax.lax.dot_general(h.astype(jnp.bfloat16), wd_ref[...],
                                (((1,), (1,)), ((), ())),
                                preferred_element_type=jnp.float32)
        sel = jnp.sum(jnp.where(lane == e, wfull_ref[rows, :], 0.0),
                      axis=-1, keepdims=True)
        w = jnp.where(e == E, 1.0, sel)
        out_ref[rows, :] += w * o


@jax.jit
def kernel(hidden_states, gate_w, w1_gate, w1_up, w2, sw_gate, sw_up, sw_down):
    grid = (E + 1,)
    out = pl.pallas_call(
        _moe_body,
        grid=grid,
        in_specs=[
            pl.BlockSpec((T, D), lambda e: (0, 0)),
            pl.BlockSpec((E, D), lambda e: (0, 0)),
            pl.BlockSpec((1, DFF, D), lambda e: (jnp.minimum(e, E - 1), 0, 0)),
            pl.BlockSpec((1, DFF, D), lambda e: (jnp.minimum(e, E - 1), 0, 0)),
            pl.BlockSpec((1, D, DFF), lambda e: (jnp.minimum(e, E - 1), 0, 0)),
            pl.BlockSpec((DFF, D), lambda e: (0, 0)),
            pl.BlockSpec((DFF, D), lambda e: (0, 0)),
            pl.BlockSpec((D, DFF), lambda e: (0, 0)),
        ],
        out_specs=pl.BlockSpec((T, D), lambda e: (0, 0)),
        out_shape=jax.ShapeDtypeStruct((T, D), jnp.float32),
        scratch_shapes=[
            pltpu.VMEM((T, D), jnp.bfloat16),
            pltpu.VMEM((T, E), jnp.float32),
            pltpu.VMEM((2 * DFF, D), jnp.bfloat16),
            pltpu.VMEM((D, DFF), jnp.bfloat16),
        ],
        compiler_params=pltpu.CompilerParams(
            dimension_semantics=("arbitrary",)),
    )(hidden_states, gate_w, w1_gate, w1_up, w2, sw_gate, sw_up, sw_down)
    return out
